# baseline (speedup 1.0000x reference)
import functools

import jax
import jax.numpy as jnp
from jax import lax
from jax.experimental import pallas as pl
from jax.experimental.pallas import tpu as pltpu
from jax.experimental.pallas import tpu_sc as plsc

NUM_WORKERS = 32
STRIPE = 128
DIM = 64
NBUF = 8
B0 = 16384
B1 = 50


@functools.lru_cache(maxsize=None)
def _make(n_stripes):
  assert n_stripes % NBUF == 0
  n_groups = n_stripes // NBUF
  s_per_b1 = B0 // STRIPE
  mesh = plsc.VectorSubcoreMesh(core_axis_name="c", subcore_axis_name="s")

  @functools.partial(
      pl.kernel,
      mesh=mesh,
      compiler_params=pltpu.CompilerParams(use_tc_tiling_on_sc=False),
      out_type=jax.ShapeDtypeStruct((B1, B0, 2 * DIM), jnp.float32),
      scratch_types=[
          pltpu.VMEM((n_stripes, STRIPE), jnp.int32),
          pltpu.VMEM((NBUF, STRIPE, DIM), jnp.float32),
          pltpu.SemaphoreType.DMA((NBUF,)),
          pltpu.SemaphoreType.DMA((NBUF,)),
      ],
  )
  def gather_kernel(idx_hbm, table_hbm, out_hbm, idx_v, rows_v, gsem, wsem):
    wid = lax.axis_index("s") * 2 + lax.axis_index("c")
    base_st = wid * n_stripes
    pltpu.sync_copy(idx_hbm.at[wid], idx_v)

    def dst_slice(c):
      st = base_st + c
      b1 = st // s_per_b1
      b0s = (st % s_per_b1) * STRIPE
      return out_hbm.at[b1, pl.ds(b0s, STRIPE), pl.ds(0, DIM)]

    for b in range(NBUF):
      pltpu.async_copy(table_hbm.at[idx_v.at[b]], rows_v.at[b], gsem.at[b])

    def group(g, carry):
      for b in range(NBUF):
        c = g * NBUF + b
        pltpu.make_async_copy(table_hbm.at[idx_v.at[c]], rows_v.at[b],
                              gsem.at[b]).wait()
        pltpu.async_copy(rows_v.at[b], dst_slice(c), wsem.at[b])
      for b in range(NBUF):
        c = g * NBUF + b
        pltpu.make_async_copy(rows_v.at[b], dst_slice(c), wsem.at[b]).wait()

        @pl.when(c + NBUF < n_stripes)
        def _():
          pltpu.async_copy(table_hbm.at[idx_v.at[c + NBUF]], rows_v.at[b],
                           gsem.at[b])

      return carry

    lax.fori_loop(0, n_groups, group, 0)

  return gather_kernel


def _tc_relayout(tt):
  def body(x_ref, o_ref):
    t = x_ref[...].T
    e = t.reshape(t.shape[0] // 2, 2, DIM)
    o_ref[:, 0:DIM] = e[:, 0, :]
    o_ref[:, DIM:2 * DIM] = e[:, 1, :]

  n_cols = tt.shape[1]
  blk = 512
  grid = (n_cols + blk - 1) // blk
  return pl.pallas_call(
      body,
      grid=(grid,),
      in_specs=[pl.BlockSpec((DIM, blk), lambda i: (0, i))],
      out_specs=pl.BlockSpec((blk // 2, 2 * DIM), lambda i: (i, 0)),
      out_shape=jax.ShapeDtypeStruct((n_cols // 2, 2 * DIM), jnp.float32),
  )(tt)


def kernel(indices, table):
  b0, b1 = indices.shape
  n_stripes = b0 * b1 // (NUM_WORKERS * STRIPE)
  idx = indices.astype(jnp.int32).T.reshape(NUM_WORKERS, n_stripes, STRIPE)
  t2 = _tc_relayout(table.T).reshape(table.shape)
  out = _make(n_stripes)(idx, t2)
  return jnp.swapaxes(out[:, :, :DIM], 0, 1)


# speedup vs baseline: 1.8609x; 1.8609x over previous
import functools

import jax
import jax.numpy as jnp
from jax import lax
from jax.experimental import pallas as pl
from jax.experimental.pallas import tpu as pltpu
from jax.experimental.pallas import tpu_sc as plsc

NUM_WORKERS = 32
STRIPE = 128
DIM = 64
NBUF = 8
B0 = 16384
B1 = 50


@functools.lru_cache(maxsize=None)
def _make(n_stripes):
  assert n_stripes % NBUF == 0
  n_groups = n_stripes // NBUF
  s_per_b1 = B0 // STRIPE
  mesh = plsc.VectorSubcoreMesh(core_axis_name="c", subcore_axis_name="s")

  @functools.partial(
      pl.kernel,
      mesh=mesh,
      compiler_params=pltpu.CompilerParams(use_tc_tiling_on_sc=False),
      out_type=jax.ShapeDtypeStruct((B1, B0, 2 * DIM), jnp.float32),
      scratch_types=[
          pltpu.VMEM((n_stripes, STRIPE), jnp.int32),
          pltpu.VMEM((NBUF, STRIPE, DIM), jnp.float32),
          pltpu.SemaphoreType.DMA((NBUF,)),
          pltpu.SemaphoreType.DMA((NBUF,)),
      ],
  )
  def gather_kernel(idx_hbm, table_hbm, out_hbm, idx_v, rows_v, gsem, wsem):
    wid = lax.axis_index("s") * 2 + lax.axis_index("c")
    base_st = wid * n_stripes
    pltpu.sync_copy(idx_hbm.at[wid], idx_v)

    def dst_slice(c):
      st = base_st + c
      b1 = st // s_per_b1
      b0s = (st % s_per_b1) * STRIPE
      return out_hbm.at[b1, pl.ds(b0s, STRIPE), pl.ds(0, DIM)]

    for b in range(NBUF):
      pltpu.async_copy(table_hbm.at[idx_v.at[b]], rows_v.at[b], gsem.at[b])

    def group(g, carry):
      for b in range(NBUF):
        c = g * NBUF + b
        pltpu.make_async_copy(table_hbm.at[idx_v.at[c]], rows_v.at[b],
                              gsem.at[b]).wait()
        pltpu.async_copy(rows_v.at[b], dst_slice(c), wsem.at[b])
      for b in range(NBUF):
        c = g * NBUF + b
        pltpu.make_async_copy(rows_v.at[b], dst_slice(c), wsem.at[b]).wait()

        @pl.when(c + NBUF < n_stripes)
        def _():
          pltpu.async_copy(table_hbm.at[idx_v.at[c + NBUF]], rows_v.at[b],
                           gsem.at[b])

      return carry

    lax.fori_loop(0, n_groups, group, 0)

  return gather_kernel


def _tc_relayout(tt):
  def body(x_ref, o_ref):
    t = x_ref[...].T
    e = t.reshape(t.shape[0] // 2, 2, DIM)
    o_ref[:, 0:DIM] = e[:, 0, :]
    o_ref[:, DIM:2 * DIM] = e[:, 1, :]

  n_cols = tt.shape[1]
  blk = 4096
  grid = (n_cols + blk - 1) // blk
  return pl.pallas_call(
      body,
      grid=(grid,),
      in_specs=[pl.BlockSpec((DIM, blk), lambda i: (0, i))],
      out_specs=pl.BlockSpec((blk // 2, 2 * DIM), lambda i: (i, 0)),
      out_shape=jax.ShapeDtypeStruct((n_cols // 2, 2 * DIM), jnp.float32),
  )(tt)


def kernel(indices, table):
  b0, b1 = indices.shape
  n_stripes = b0 * b1 // (NUM_WORKERS * STRIPE)
  idx = indices.astype(jnp.int32).T.reshape(NUM_WORKERS, n_stripes, STRIPE)
  t2 = _tc_relayout(table.T).reshape(table.shape)
  out = _make(n_stripes)(idx, t2)
  return jnp.swapaxes(out[:, :, :DIM], 0, 1)


# speedup vs baseline: 1.9395x; 1.0423x over previous
import functools

import jax
import jax.numpy as jnp
from jax import lax
from jax.experimental import pallas as pl
from jax.experimental.pallas import tpu as pltpu
from jax.experimental.pallas import tpu_sc as plsc

NUM_WORKERS = 32
STRIPE = 128
DIM = 64
NBUF = 8
B0 = 16384
B1 = 50


@functools.lru_cache(maxsize=None)
def _make(n_stripes):
  assert n_stripes % NBUF == 0
  n_groups = n_stripes // NBUF
  s_per_b1 = B0 // STRIPE
  mesh = plsc.VectorSubcoreMesh(core_axis_name="c", subcore_axis_name="s")

  @functools.partial(
      pl.kernel,
      mesh=mesh,
      compiler_params=pltpu.CompilerParams(use_tc_tiling_on_sc=False),
      out_type=jax.ShapeDtypeStruct((B1, B0, 2 * DIM), jnp.float32),
      scratch_types=[
          pltpu.VMEM((n_stripes, STRIPE), jnp.int32),
          pltpu.VMEM((NBUF, STRIPE, DIM), jnp.float32),
          pltpu.SemaphoreType.DMA((NBUF,)),
          pltpu.SemaphoreType.DMA((NBUF,)),
      ],
  )
  def gather_kernel(idx_hbm, table_hbm, out_hbm, idx_v, rows_v, gsem, wsem):
    wid = lax.axis_index("s") * 2 + lax.axis_index("c")
    base_st = wid * n_stripes
    pltpu.sync_copy(idx_hbm.at[wid], idx_v)

    def dst_slice(c):
      st = base_st + c
      b1 = st // s_per_b1
      b0s = (st % s_per_b1) * STRIPE
      return out_hbm.at[b1, pl.ds(b0s, STRIPE), pl.ds(0, DIM)]

    for b in range(NBUF):
      pltpu.async_copy(table_hbm.at[idx_v.at[b]], rows_v.at[b], gsem.at[b])

    def group(g, carry):
      for b in range(NBUF):
        c = g * NBUF + b
        pltpu.make_async_copy(table_hbm.at[idx_v.at[c]], rows_v.at[b],
                              gsem.at[b]).wait()
        pltpu.async_copy(rows_v.at[b], dst_slice(c), wsem.at[b])
      for b in range(NBUF):
        c = g * NBUF + b
        pltpu.make_async_copy(rows_v.at[b], dst_slice(c), wsem.at[b]).wait()

        @pl.when(c + NBUF < n_stripes)
        def _():
          pltpu.async_copy(table_hbm.at[idx_v.at[c + NBUF]], rows_v.at[b],
                           gsem.at[b])

      return carry

    lax.fori_loop(0, n_groups, group, 0)

  return gather_kernel


def _tc_relayout(tt):
  def body(x_ref, o_ref):
    t = x_ref[...].T
    e = t.reshape(t.shape[0] // 2, 2, DIM)
    o_ref[:, 0:DIM] = e[:, 0, :]
    o_ref[:, DIM:2 * DIM] = e[:, 1, :]

  n_cols = tt.shape[1]
  blk = 16384
  grid = (n_cols + blk - 1) // blk
  return pl.pallas_call(
      body,
      grid=(grid,),
      in_specs=[pl.BlockSpec((DIM, blk), lambda i: (0, i))],
      out_specs=pl.BlockSpec((blk // 2, 2 * DIM), lambda i: (i, 0)),
      out_shape=jax.ShapeDtypeStruct((n_cols // 2, 2 * DIM), jnp.float32),
  )(tt)


def kernel(indices, table):
  b0, b1 = indices.shape
  n_stripes = b0 * b1 // (NUM_WORKERS * STRIPE)
  idx = indices.astype(jnp.int32).T.reshape(NUM_WORKERS, n_stripes, STRIPE)
  t2 = _tc_relayout(table.T).reshape(table.shape)
  out = _make(n_stripes)(idx, t2)
  return jnp.swapaxes(out[:, :, :DIM], 0, 1)
